# Initial kernel scaffold; baseline (speedup 1.0000x reference)
import jax
import jax.numpy as jnp
from jax.experimental import pallas as pl


def kernel(x, edge_index, edge_weight, params):
    raise NotImplementedError("write your pallas kernel here")


# speedup vs baseline: 2.1631x; 2.1631x over previous
import functools

import jax
import jax.numpy as jnp
from jax import lax
from jax.experimental import pallas as pl
from jax.experimental.pallas import tpu as pltpu
from jax.experimental.pallas import tpu_sc as plsc

N = 10000
N_PAD = 10240
E = 160000
NS = 16
NC = 2
EB = 128
NBAT = 80
E_PAD = NS * NBAT * EB
RPT = N_PAD // NS

_DIMS = [(128, 500), (500, 1000)] + [(1000, 1000)] * 8 + [(1000, 500), (500, 128)]


def _cdiv(a, b):
    return (a + b - 1) // b


def _mesh():
    return plsc.VectorSubcoreMesh(core_axis_name="c", subcore_axis_name="s")


def _fill_const(ref, val):
    def row(i, _):
        for q in range(8):
            ref[i, pl.ds(q * 16, 16)] = jnp.full((16,), val, jnp.float32)
        return 0
    lax.fori_loop(0, EB, row, 0)


@functools.lru_cache(maxsize=None)
def _make_deg():
    @functools.partial(
        pl.kernel,
        out_type=jax.ShapeDtypeStruct((N_PAD, 128), jnp.float32),
        mesh=_mesh(),
        scratch_types=[
            pltpu.VMEM((NBAT, EB), jnp.int32),
            pltpu.VMEM((EB, 128), jnp.float32),
            pltpu.VMEM((EB, 128), jnp.float32),
            pltpu.VMEM_SHARED((N_PAD, 128), jnp.float32),
        ],
    )
    def degk(dst_hbm, out_hbm, dst_v, ones_v, zbuf, acc):
        core = lax.axis_index("c")
        s = lax.axis_index("s")

        @pl.when(core == 0)
        def _():
            pltpu.sync_copy(dst_hbm.at[s], dst_v)
            _fill_const(ones_v, 1.0)
            _fill_const(zbuf, 0.0)
            for p in range(RPT // EB):
                pltpu.sync_copy(zbuf, acc.at[pl.ds(s * RPT + p * EB, EB)])
            plsc.subcore_barrier()

            def step(j, _):
                pltpu.sync_copy(ones_v, acc.at[dst_v.at[j]], add=True)
                return 0
            lax.fori_loop(0, NBAT, step, 0)
            plsc.subcore_barrier()
            pltpu.sync_copy(acc.at[pl.ds(s * RPT, RPT)],
                            out_hbm.at[pl.ds(s * RPT, RPT)])

    return degk


HNB = NBAT // 2


@functools.lru_cache(maxsize=None)
def _make_agg(C):
    n_k = (C + 1) // 2

    @functools.partial(
        pl.kernel,
        out_type=jax.ShapeDtypeStruct((C, N_PAD, 128), jnp.float32),
        mesh=_mesh(),
        scratch_types=[
            pltpu.VMEM((HNB, EB), jnp.int32),
            pltpu.VMEM((HNB, EB), jnp.int32),
            pltpu.VMEM((EB, 128), jnp.float32),
            pltpu.VMEM((EB, 128), jnp.float32),
            pltpu.VMEM_SHARED((N_PAD, 128), jnp.float32),
            pltpu.SemaphoreType.DMA,
            pltpu.SemaphoreType.DMA,
            pltpu.SemaphoreType.DMA,
            pltpu.SemaphoreType.DMA,
        ],
    )
    def aggk(h_hbm, src_hbm, dst_hbm, out_hbm,
             src_v, dst_v, buf0, buf1, acc, g0, g1, s0, s1):
        core = lax.axis_index("c")
        s = lax.axis_index("s")
        bufs = (buf0, buf1)
        gsems = (g0, g1)
        ssems = (s0, s1)

        for k in range(n_k):
            chunk = 2 * k + core

            def run_chunk(chunk=chunk):
                table = h_hbm.at[chunk]
                _fill_const(buf0, 0.0)
                for p in range(RPT // EB):
                    pltpu.sync_copy(buf0, acc.at[pl.ds(s * RPT + p * EB, EB)])
                plsc.subcore_barrier()

                for half in range(2):
                    pltpu.sync_copy(src_hbm.at[s, pl.ds(half * HNB, HNB)], src_v)
                    pltpu.sync_copy(dst_hbm.at[s, pl.ds(half * HNB, HNB)], dst_v)
                    for b in range(2):
                        pltpu.async_copy(table.at[src_v.at[b]], bufs[b], gsems[b])

                    def step(it, _):
                        for b in range(2):
                            j = 2 * it + b
                            pltpu.make_async_copy(
                                table.at[src_v.at[j]], bufs[b], gsems[b]).wait()
                            pltpu.async_copy(
                                bufs[b], acc.at[dst_v.at[j]], ssems[b], add=True)
                            pltpu.make_async_copy(
                                bufs[b], acc.at[dst_v.at[j]], ssems[b]).wait()

                            @pl.when(j + 2 < HNB)
                            def _():
                                pltpu.async_copy(
                                    table.at[src_v.at[j + 2]], bufs[b], gsems[b])
                        return 0
                    lax.fori_loop(0, HNB // 2, step, 0)

                plsc.subcore_barrier()
                pltpu.sync_copy(acc.at[pl.ds(s * RPT, RPT)],
                                out_hbm.at[chunk, pl.ds(s * RPT, RPT)])
                plsc.subcore_barrier()

            if C % 2 == 1:
                pl.when(chunk < C)(run_chunk)
            else:
                run_chunk()

    return aggk


@functools.lru_cache(maxsize=None)
def _make_conv(Ci, Co, relu):
    NT = 8
    MT = N_PAD // NT

    def body(m_ref, deg_ref, h_ref, wl_ref, wr_ref, b_ref, out_ref):
        ci = pl.program_id(1)
        inv = 1.0 / jnp.maximum(deg_ref[...], 1.0)
        mh = m_ref[0] * inv
        ht = h_ref[0]
        for co in range(Co):
            p = (jnp.dot(mh, wl_ref[0, co], preferred_element_type=jnp.float32)
                 + jnp.dot(ht, wr_ref[0, co], preferred_element_type=jnp.float32))

            @pl.when(ci == 0)
            def _(p=p, co=co):
                out_ref[co] = p + b_ref[co][None, :]

            @pl.when(ci > 0)
            def _(p=p, co=co):
                out_ref[co] += p

        if relu:
            @pl.when(ci == Ci - 1)
            def _():
                out_ref[...] = jnp.maximum(out_ref[...], 0.0)

    return pl.pallas_call(
        body,
        grid=(NT, Ci),
        in_specs=[
            pl.BlockSpec((1, MT, 128), lambda nt, ci: (ci, nt, 0)),
            pl.BlockSpec((MT, 128), lambda nt, ci: (nt, 0)),
            pl.BlockSpec((1, MT, 128), lambda nt, ci: (ci, nt, 0)),
            pl.BlockSpec((1, Co, 128, 128), lambda nt, ci: (ci, 0, 0, 0)),
            pl.BlockSpec((1, Co, 128, 128), lambda nt, ci: (ci, 0, 0, 0)),
            pl.BlockSpec((Co, 128), lambda nt, ci: (0, 0)),
        ],
        out_specs=pl.BlockSpec((Co, MT, 128), lambda nt, ci: (0, nt, 0)),
        out_shape=jax.ShapeDtypeStruct((Co, N_PAD, 128), jnp.float32),
    )


def _prep_w(W, Ci, Co):
    Wp = jnp.zeros((Ci * 128, Co * 128), jnp.float32)
    Wp = Wp.at[:W.shape[0], :W.shape[1]].set(W)
    return Wp.reshape(Ci, 128, Co, 128).transpose(0, 2, 1, 3)


def kernel(x, edge_index, edge_weight, params):
    del edge_weight
    src = edge_index[0].astype(jnp.int32)
    dst = edge_index[1].astype(jnp.int32)
    pad = E_PAD - E
    src_p = jnp.concatenate([src, jnp.zeros((pad,), jnp.int32)]).reshape(NS, NBAT, EB)
    dst_p = jnp.concatenate([dst, jnp.full((pad,), N_PAD - 1, jnp.int32)]).reshape(NS, NBAT, EB)

    deg = _make_deg()(dst_p)

    h = jnp.zeros((1, N_PAD, 128), jnp.float32).at[0, :N, :].set(x)

    conv_dims = []
    for (din, dout) in _DIMS:
        conv_dims += [(din, dout), (dout, dout)]

    n_convs = len(conv_dims)
    for i, (din, dout) in enumerate(conv_dims):
        Ci, Co = _cdiv(din, 128), _cdiv(dout, 128)
        W_l, b_l, W_r = params[i]
        wl = _prep_w(W_l, Ci, Co)
        wr = _prep_w(W_r, Ci, Co)
        bb = jnp.zeros((Co * 128,), jnp.float32).at[:dout].set(b_l).reshape(Co, 128)
        m = _make_agg(Ci)(h, src_p, dst_p)
        h = _make_conv(Ci, Co, i < n_convs - 1)(m, deg, h, wl, wr, bb)

    return h[0, :N, :]
